# baseline (speedup 1.0000x reference)
import functools

import jax
import jax.numpy as jnp
from jax import lax
from jax.experimental import pallas as pl
from jax.experimental.pallas import tpu as pltpu
from jax.experimental.pallas import tpu_sc as plsc

BATCH = 4096
F = 26
E = 16
A = 16
P = F * (F - 1) // 2
S = (P + 15) // 16
P_PAD = S * 16

NC, NS = 2, 16
NW = NC * NS
N_LOOKUPS = BATCH * F
PER_W = N_LOOKUPS // NW
CHUNK = 128
NCHUNK = PER_W // CHUNK

BBLK = 128
NBLK = BATCH // BBLK


def _sc_body(idx_hbm, emb_hbm, fow_hbm, out_emb, out_fow,
             idx_v, rows_v, fow_v, sem_e, sem_f):
    wid = lax.axis_index("s") * NC + lax.axis_index("c")
    pltpu.sync_copy(idx_hbm.at[wid], idx_v)

    def fire(k, carry):
        pltpu.async_copy(emb_hbm.at[idx_v.at[k]], rows_v.at[k], sem_e)
        pltpu.async_copy(fow_hbm.at[idx_v.at[k]], fow_v.at[k], sem_f)
        return carry

    lax.fori_loop(0, NCHUNK, fire, 0)

    def drain(k, carry):
        pltpu.make_async_copy(emb_hbm.at[idx_v.at[0]], rows_v.at[0], sem_e).wait()
        pltpu.make_async_copy(fow_hbm.at[idx_v.at[0]], fow_v.at[0], sem_f).wait()
        return carry

    lax.fori_loop(0, NCHUNK, drain, 0)

    pltpu.sync_copy(rows_v, out_emb.at[wid])
    pltpu.sync_copy(fow_v, out_fow.at[wid])


@functools.cache
def _sc_gather_fn():
    mesh = plsc.VectorSubcoreMesh(
        core_axis_name="c", subcore_axis_name="s",
        num_cores=NC, num_subcores=NS)
    return pl.kernel(
        _sc_body,
        out_type=(
            jax.ShapeDtypeStruct((NW, NCHUNK, CHUNK, E), jnp.float32),
            jax.ShapeDtypeStruct((NW, NCHUNK, CHUNK, 1), jnp.float32),
        ),
        mesh=mesh,
        scratch_types=[
            pltpu.VMEM((NCHUNK, CHUNK), jnp.int32),
            pltpu.VMEM((NCHUNK, CHUNK, E), jnp.float32),
            pltpu.VMEM((NCHUNK, CHUNK, 1), jnp.float32),
            pltpu.SemaphoreType.DMA,
            pltpu.SemaphoreType.DMA,
        ],
        compiler_params=pltpu.CompilerParams(use_tc_tiling_on_sc=False),
    )


def _tc_body(embT_ref, fvT_ref, fowT_ref, bd_ref, bb_ref, hbd_ref,
             pp_ref, bias_ref, out_ref, bi_ref, log_ref):
    fv = fvT_ref[...]
    ev = embT_ref[...] * fv[:, None, :]

    off = 0
    for i in range(F - 1):
        n = F - 1 - i
        bi_ref[off:off + n] = ev[i + 1:F] * ev[i:i + 1]
        off += n
    bi_ref[P:P_PAD] = jnp.zeros((P_PAD - P, E, BBLK), jnp.float32)

    bd = bd_ref[...]
    bb = bb_ref[...]
    hbd = hbd_ref[...]
    for s in range(S):
        bi_s = bi_ref[s * 16:(s + 1) * 16].reshape(16 * E, BBLK)
        att = jnp.maximum(
            jnp.dot(bd, bi_s, preferred_element_type=jnp.float32) + bb, 0.0)
        log_ref[s * 16:(s + 1) * 16] = jnp.dot(
            hbd, att, preferred_element_type=jnp.float32)
    log_ref[P:P_PAD] = jnp.full((P_PAD - P, BBLK), -1e30, jnp.float32)

    logits = log_ref[...]
    m = jnp.max(logits, axis=0, keepdims=True)
    ex = jnp.exp(logits - m)
    z = jnp.sum(ex, axis=0, keepdims=True)
    score = ex / z

    aw = jnp.sum(score[:, None, :] * bi_ref[...], axis=0)
    awp = jnp.sum(aw * pp_ref[...], axis=0, keepdims=True)
    y1 = jnp.sum(fowT_ref[...] * fv, axis=0, keepdims=True)
    y = bias_ref[...] + y1 + awp
    out_ref[...] = (1.0 / (1.0 + jnp.exp(-y)))[None]


def _tc_forward(embT, fvT, fowT, bd, bb, hbd, pp, bias_r):
    return pl.pallas_call(
        _tc_body,
        grid=(NBLK,),
        in_specs=[
            pl.BlockSpec((F, E, BBLK), lambda i: (0, 0, i)),
            pl.BlockSpec((F, BBLK), lambda i: (0, i)),
            pl.BlockSpec((F, BBLK), lambda i: (0, i)),
            pl.BlockSpec((16 * A, 16 * E), lambda i: (0, 0)),
            pl.BlockSpec((16 * A, 1), lambda i: (0, 0)),
            pl.BlockSpec((16, 16 * A), lambda i: (0, 0)),
            pl.BlockSpec((E, 1), lambda i: (0, 0)),
            pl.BlockSpec((1, 1), lambda i: (0, 0)),
        ],
        out_specs=pl.BlockSpec((1, 1, BBLK), lambda i: (i, 0, 0)),
        out_shape=jax.ShapeDtypeStruct((NBLK, 1, BBLK), jnp.float32),
        scratch_shapes=[
            pltpu.VMEM((P_PAD, E, BBLK), jnp.float32),
            pltpu.VMEM((P_PAD, BBLK), jnp.float32),
        ],
    )(embT, fvT, fowT, bd, bb, hbd, pp, bias_r)


def kernel(feat_index, feat_value, first_order_w, emb_table, bias,
           attention_w, attention_b, projection_h, projection_p):
    idx = feat_index.astype(jnp.int32).T.reshape(NW, NCHUNK, CHUNK)
    emb_fb, fow_fb = _sc_gather_fn()(idx, emb_table, first_order_w)

    embT = emb_fb.reshape(F, BATCH, E).transpose(0, 2, 1)
    fowT = fow_fb.reshape(F, BATCH)
    fvT = feat_value.T

    eye = jnp.eye(16, dtype=jnp.float32)
    bd = jnp.kron(eye, attention_w.T)
    bb = jnp.tile(attention_b, 16)[:, None]
    hbd = jnp.kron(eye, projection_h[:, 0][None, :])
    bias_r = bias.reshape(1, 1)

    out = _tc_forward(embT, fvT, fowT, bd, bb, hbd, projection_p, bias_r)
    return out.reshape(BATCH)
